# baseline (speedup 1.0000x reference)
import functools

import jax
import jax.numpy as jnp
import numpy as np
from jax import lax
from jax.experimental import pallas as pl
from jax.experimental.pallas import tpu as pltpu
from jax.experimental.pallas import tpu_sc as plsc

N_EMB = 1000000
D_EMB = 64
BATCH = 16384

NC = 2
NS = 16
L = 16
NW = NC * NS
B_PER_W = BATCH // NW
N_CHUNK = B_PER_W // 128

BLK = 16384
N_BLK = (N_EMB + BLK - 1) // BLK
Z_LEN = N_BLK * BLK


def _mv_body(p_ref, w_ref, b_ref, z_ref):
    z = lax.dot_general(w_ref[...], p_ref[...], (((1,), (0,)), ((), ())),
                        preferred_element_type=jnp.float32)
    z_ref[...] = z.reshape(BLK) + b_ref[0, 0]


@jax.jit
def _tc_matvec(p, w, b):
    return pl.pallas_call(
        _mv_body,
        grid=(N_BLK,),
        in_specs=[
            pl.BlockSpec((D_EMB, BLK), lambda c: (0, c)),
            pl.BlockSpec((1, D_EMB), lambda c: (0, 0)),
            pl.BlockSpec((1, 1), lambda c: (0, 0)),
        ],
        out_specs=pl.BlockSpec((BLK,), lambda c: (c,)),
        out_shape=jax.ShapeDtypeStruct((Z_LEN,), jnp.float32),
    )(p, w, b)


def _sc_body(idx_hbm, z_hbm, out_hbm, idx_v, out_v, z_sh, sem):
    cid = lax.axis_index("c")
    sid = lax.axis_index("s")
    wid = sid * NC + cid
    base = wid * B_PER_W

    @pl.when(sid == 0)
    def _():
        pltpu.sync_copy(z_hbm, z_sh)

    plsc.subcore_barrier()

    pltpu.sync_copy(idx_hbm.at[wid], idx_v)
    copies = []
    for j in range(N_CHUNK):
        copies.append(pltpu.async_copy(
            z_sh.at[idx_v.at[j]],
            out_v.at[pl.ds(j * 128, 128)],
            sem))
    for c in copies:
        c.wait()
    pltpu.sync_copy(out_v, out_hbm.at[pl.ds(base, B_PER_W)])


@jax.jit
def _sc_gather(idx_r, z):
    mesh = plsc.VectorSubcoreMesh(core_axis_name="c", subcore_axis_name="s")
    k = pl.kernel(
        _sc_body,
        mesh=mesh,
        compiler_params=pltpu.CompilerParams(use_tc_tiling_on_sc=False),
        out_type=jax.ShapeDtypeStruct((BATCH,), jnp.float32),
        scratch_types=[
            pltpu.VMEM((N_CHUNK, 128), jnp.int32),
            pltpu.VMEM((B_PER_W,), jnp.float32),
            pltpu.VMEM_SHARED((Z_LEN,), jnp.float32),
            pltpu.SemaphoreType.DMA,
        ],
    )
    return k(idx_r, z)


def kernel(idx, table, W, b):
    p = table.T
    w = W.reshape(1, D_EMB).astype(jnp.float32)
    b2 = b.reshape(1, 1).astype(jnp.float32)
    z = _tc_matvec(p, w, b2)
    idx_r = idx.astype(jnp.int32).reshape(NW, N_CHUNK, 128)
    out = _sc_gather(idx_r, z)
    return out.reshape(BATCH, 1)


# speedup vs baseline: 1.1602x; 1.1602x over previous
import functools

import jax
import jax.numpy as jnp
import numpy as np
from jax import lax
from jax.experimental import pallas as pl
from jax.experimental.pallas import tpu as pltpu
from jax.experimental.pallas import tpu_sc as plsc

N_EMB = 1000000
D_EMB = 64
BATCH = 16384

NC = 2
NS = 16
L = 16
NW = NC * NS
B_PER_W = BATCH // NW
N_CHUNK = B_PER_W // 128

BLK = 32768
N_BLK = (N_EMB + BLK - 1) // BLK
Z_LEN = N_BLK * BLK


def _mv_body(p_ref, w_ref, b_ref, z_ref):
    z = lax.dot_general(w_ref[...], p_ref[...], (((1,), (0,)), ((), ())),
                        preferred_element_type=jnp.float32)
    z_ref[...] = z.reshape(BLK) + b_ref[0, 0]


@jax.jit
def _tc_matvec(p, w, b):
    return pl.pallas_call(
        _mv_body,
        grid=(N_BLK,),
        in_specs=[
            pl.BlockSpec((D_EMB, BLK), lambda c: (0, c)),
            pl.BlockSpec((1, D_EMB), lambda c: (0, 0)),
            pl.BlockSpec((1, 1), lambda c: (0, 0)),
        ],
        out_specs=pl.BlockSpec((BLK,), lambda c: (c,)),
        out_shape=jax.ShapeDtypeStruct((Z_LEN,), jnp.float32),
    )(p, w, b)


def _sc_body(idx_hbm, z_hbm, out_hbm, idx_v, out_v, sem):
    cid = lax.axis_index("c")
    sid = lax.axis_index("s")
    wid = sid * NC + cid
    base = wid * B_PER_W

    pltpu.sync_copy(idx_hbm.at[wid], idx_v)
    copies = []
    for j in range(N_CHUNK):
        copies.append(pltpu.async_copy(
            z_hbm.at[idx_v.at[j]],
            out_v.at[pl.ds(j * 128, 128)],
            sem))
    for c in copies:
        c.wait()
    pltpu.sync_copy(out_v, out_hbm.at[pl.ds(base, B_PER_W)])


@jax.jit
def _sc_gather(idx_r, z):
    mesh = plsc.VectorSubcoreMesh(core_axis_name="c", subcore_axis_name="s")
    k = pl.kernel(
        _sc_body,
        mesh=mesh,
        compiler_params=pltpu.CompilerParams(use_tc_tiling_on_sc=False),
        out_type=jax.ShapeDtypeStruct((BATCH,), jnp.float32),
        scratch_types=[
            pltpu.VMEM((N_CHUNK, 128), jnp.int32),
            pltpu.VMEM((B_PER_W,), jnp.float32),
            pltpu.SemaphoreType.DMA,
        ],
    )
    return k(idx_r, z)


def kernel(idx, table, W, b):
    p = table.T
    w = W.reshape(1, D_EMB).astype(jnp.float32)
    b2 = b.reshape(1, 1).astype(jnp.float32)
    z = _tc_matvec(p, w, b2)
    idx_r = idx.astype(jnp.int32).reshape(NW, N_CHUNK, 128)
    out = _sc_gather(idx_r, z)
    return out.reshape(BATCH, 1)
